# baseline (speedup 1.0000x reference)
import functools

import jax
import jax.numpy as jnp
from jax.experimental import pallas as pl
from jax.experimental.pallas import tpu as pltpu

_N = 2048
_H = 8
_HD = 32
_F = 256
_C = 32
_BI = 2048
_NBLK = _N // _BI


def _scores(mask, ber, eneg, der):
    return jnp.where(mask, jnp.maximum(ber, eneg * der), jnp.bfloat16(0.0))


def _body(x_ref, w1_ref, al_ref, ar_ref, adj_ref, w2_ref, a2l_ref, a2r_ref,
          out_ref, gaug, el1s, er1t, g2aug, el2s, er2t):
    i = pl.program_id(0)
    f32 = jnp.float32
    bf16 = jnp.bfloat16

    @pl.when(i == 0)
    def _proj():
        g = jnp.dot(x_ref[...], w1_ref[...], preferred_element_type=f32)
        ones = jnp.ones((_N, 1), dtype=bf16)
        zeros = jnp.zeros((_N, 128 - _HD - 1), dtype=bf16)
        parts = []
        for h in range(_H):
            parts += [g[:, h * _HD:(h + 1) * _HD].astype(bf16), ones, zeros]
        gaug[...] = jnp.concatenate(parts, axis=1)
        el1s[...] = jnp.dot(g, al_ref[...], preferred_element_type=f32)
        er1t[...] = jnp.dot(g, ar_ref[...], preferred_element_type=f32).T

    @pl.when((i >= 1) & (i <= _NBLK))
    def _layer1():
        r0 = (i - 1) * _BI
        mask = adj_ref[...] != 0
        el = el1s[pl.ds(r0, _BI), :]
        ert = er1t[...]
        eneg = jnp.exp(-0.8 * el).astype(bf16)
        ber = jnp.exp(ert).astype(bf16)
        der = jnp.exp(0.2 * ert).astype(bf16)
        parts = []
        for h in range(_H):
            p = _scores(mask, ber[h:h + 1, :], eneg[:, h:h + 1],
                        der[h:h + 1, :])
            nd = jnp.dot(p, gaug[:, h * 128:(h + 1) * 128],
                         preferred_element_type=f32)
            parts.append(nd[:, :_HD] / nd[:, _HD:_HD + 1])
        hcat = jnp.concatenate(parts, axis=1)
        hact = jnp.where(hcat > 0, hcat,
                         jnp.exp(jnp.minimum(hcat, 0.0)) - 1.0)
        g2 = jnp.dot(hact, w2_ref[...], preferred_element_type=f32)
        g2aug[pl.ds(r0, _BI), :] = jnp.concatenate(
            [g2.astype(bf16), jnp.ones((_BI, 1), bf16),
             jnp.zeros((_BI, 64 - _C - 1), bf16)], axis=1)
        el2s[pl.ds(r0, _BI), :] = jnp.dot(g2, a2l_ref[...],
                                          preferred_element_type=f32)
        er2t[:, pl.ds(r0, _BI)] = jnp.dot(g2, a2r_ref[...],
                                          preferred_element_type=f32).T

    @pl.when(i > _NBLK)
    def _layer2():
        r0 = (i - 1 - _NBLK) * _BI
        mask = adj_ref[...] != 0
        el2 = el2s[pl.ds(r0, _BI), :]
        er2 = er2t[...]
        p = _scores(mask, jnp.exp(er2).astype(bf16),
                    jnp.exp(-0.8 * el2).astype(bf16),
                    jnp.exp(0.2 * er2).astype(bf16))
        nd = jnp.dot(p, g2aug[...], preferred_element_type=f32)
        out_ref[...] = nd[:, :_C] / nd[:, _C:_C + 1]


@functools.partial(jax.jit, static_argnames=())
def kernel(x, adj_mat, W1, a1_l, a1_r, W2, a2_l, a2_r):
    f32 = jnp.float32
    adj = adj_mat.reshape(_N, _N).astype(jnp.int8)

    eye = jnp.eye(_H, dtype=f32)
    A1l = jnp.kron(eye, a1_l.astype(f32)[:, None])
    A1r = jnp.kron(eye, a1_r.astype(f32)[:, None])

    blkmap = lambda i: ((i + _NBLK - 1) % _NBLK, 0)
    const = lambda i: (0, 0)
    out = pl.pallas_call(
        _body,
        grid=(2 * _NBLK + 1,),
        in_specs=[
            pl.BlockSpec((_N, _F), const),
            pl.BlockSpec((_F, _H * _HD), const),
            pl.BlockSpec((_H * _HD, _H), const),
            pl.BlockSpec((_H * _HD, _H), const),
            pl.BlockSpec((_BI, _N), blkmap),
            pl.BlockSpec((_F, _C), const),
            pl.BlockSpec((_C, 1), const),
            pl.BlockSpec((_C, 1), const),
        ],
        out_specs=pl.BlockSpec((_BI, _C), blkmap),
        out_shape=jax.ShapeDtypeStruct((_N, _C), f32),
        scratch_shapes=[
            pltpu.VMEM((_N, _H * 128), jnp.bfloat16),
            pltpu.VMEM((_N, _H), f32),
            pltpu.VMEM((_H, _N), f32),
            pltpu.VMEM((_N, 64), jnp.bfloat16),
            pltpu.VMEM((_N, 1), f32),
            pltpu.VMEM((1, _N), f32),
        ],
    )(x, W1, A1l, A1r, adj, W2.astype(f32), a2_l.astype(f32)[:, None],
      a2_r.astype(f32)[:, None])
    return out


# speedup vs baseline: 1.0040x; 1.0040x over previous
import functools

import jax
import jax.numpy as jnp
from jax.experimental import pallas as pl
from jax.experimental.pallas import tpu as pltpu

_N = 2048
_H = 8
_HD = 32
_F = 256
_C = 32
_BI = 1024
_NBLK = _N // _BI


def _scores(maskbf, ber, eneg, der):
    return maskbf * jnp.maximum(ber, eneg * der)


def _body(x_ref, w1_ref, al_ref, ar_ref, adj_ref, w2_ref, a2l_ref, a2r_ref,
          out_ref, gaug, el1s, er1t, g2aug, el2s, er2t):
    i = pl.program_id(0)
    f32 = jnp.float32
    bf16 = jnp.bfloat16

    @pl.when(i == 0)
    def _proj():
        g = jnp.dot(x_ref[...], w1_ref[...], preferred_element_type=f32)
        ones = jnp.ones((_N, 1), dtype=bf16)
        zeros = jnp.zeros((_N, 128 - _HD - 1), dtype=bf16)
        parts = []
        for h in range(_H):
            parts += [g[:, h * _HD:(h + 1) * _HD].astype(bf16), ones, zeros]
        gaug[...] = jnp.concatenate(parts, axis=1)
        el1s[...] = jnp.dot(g, al_ref[...], preferred_element_type=f32)
        er1t[...] = jnp.dot(g, ar_ref[...], preferred_element_type=f32).T

    @pl.when((i >= 1) & (i <= _NBLK))
    def _layer1():
        r0 = (i - 1) * _BI
        maskbf = adj_ref[...].astype(bf16)
        el = el1s[pl.ds(r0, _BI), :]
        ert = er1t[...]
        eneg = jnp.exp(-0.8 * el).astype(bf16)
        ber = jnp.exp(ert).astype(bf16)
        der = jnp.exp(0.2 * ert).astype(bf16)
        parts = []
        for h in range(_H):
            p = _scores(maskbf, ber[h:h + 1, :], eneg[:, h:h + 1],
                        der[h:h + 1, :])
            nd = jnp.dot(p, gaug[:, h * 128:(h + 1) * 128],
                         preferred_element_type=f32)
            parts.append(nd[:, :_HD] / nd[:, _HD:_HD + 1])
        hcat = jnp.concatenate(parts, axis=1)
        hact = jnp.where(hcat > 0, hcat,
                         jnp.exp(jnp.minimum(hcat, 0.0)) - 1.0)
        g2 = jnp.dot(hact, w2_ref[...], preferred_element_type=f32)
        g2aug[pl.ds(r0, _BI), :] = jnp.concatenate(
            [g2.astype(bf16), jnp.ones((_BI, 1), bf16),
             jnp.zeros((_BI, 64 - _C - 1), bf16)], axis=1)
        el2s[pl.ds(r0, _BI), :] = jnp.dot(g2, a2l_ref[...],
                                          preferred_element_type=f32)
        er2t[:, pl.ds(r0, _BI)] = jnp.dot(g2, a2r_ref[...],
                                          preferred_element_type=f32).T

    @pl.when(i > _NBLK)
    def _layer2():
        r0 = (i - 1 - _NBLK) * _BI
        maskbf = adj_ref[...].astype(bf16)
        el2 = el2s[pl.ds(r0, _BI), :]
        er2 = er2t[...]
        p = _scores(maskbf, jnp.exp(er2).astype(bf16),
                    jnp.exp(-0.8 * el2).astype(bf16),
                    jnp.exp(0.2 * er2).astype(bf16))
        nd = jnp.dot(p, g2aug[...], preferred_element_type=f32)
        out_ref[...] = nd[:, :_C] / nd[:, _C:_C + 1]


@functools.partial(jax.jit, static_argnames=())
def kernel(x, adj_mat, W1, a1_l, a1_r, W2, a2_l, a2_r):
    f32 = jnp.float32
    adj = adj_mat.reshape(_N, _N).astype(jnp.int8)

    eye = jnp.eye(_H, dtype=f32)
    A1l = jnp.kron(eye, a1_l.astype(f32)[:, None])
    A1r = jnp.kron(eye, a1_r.astype(f32)[:, None])

    blkmap = lambda i: ((i + _NBLK - 1) % _NBLK, 0)
    const = lambda i: (0, 0)
    out = pl.pallas_call(
        _body,
        grid=(2 * _NBLK + 1,),
        in_specs=[
            pl.BlockSpec((_N, _F), const),
            pl.BlockSpec((_F, _H * _HD), const),
            pl.BlockSpec((_H * _HD, _H), const),
            pl.BlockSpec((_H * _HD, _H), const),
            pl.BlockSpec((_BI, _N), blkmap),
            pl.BlockSpec((_F, _C), const),
            pl.BlockSpec((_C, 1), const),
            pl.BlockSpec((_C, 1), const),
        ],
        out_specs=pl.BlockSpec((_BI, _C), blkmap),
        out_shape=jax.ShapeDtypeStruct((_N, _C), f32),
        scratch_shapes=[
            pltpu.VMEM((_N, _H * 128), jnp.bfloat16),
            pltpu.VMEM((_N, _H), f32),
            pltpu.VMEM((_H, _N), f32),
            pltpu.VMEM((_N, 64), jnp.bfloat16),
            pltpu.VMEM((_N, 1), f32),
            pltpu.VMEM((1, _N), f32),
        ],
    )(x, W1, A1l, A1r, adj, W2.astype(f32), a2_l.astype(f32)[:, None],
      a2_r.astype(f32)[:, None])
    return out
